# baseline (speedup 1.0000x reference)
import jax
import jax.numpy as jnp
from jax import lax
from jax.experimental import pallas as pl
from jax.experimental.pallas import tpu as pltpu
from jax.experimental.pallas import tpu_sc as plsc

NX = 512
NY = 512
G = NX * NY
C = 64
B = 4
P = 40000

NC = 2
NS = 16
NG = 8
CH = 1280
PBATCH = P // B
LASTN = PBATCH - (NG - 1) * CH
BBLK = NG * CH
GS = G // NG
NPL = C + 1
DUMP = jnp.int32(1 << 29)

FEAT_WORDS = B * C * G
PTS_WORDS = B * G
FTG_WORDS = B * NPL * BBLK

PB = 256
PBL = 208


def _sc_body(coords_hbm, feats_hbm, npts_hbm, fout, pout, ftg,
             crow, linb, rowb, fstage, trbuf, str0, str1,
             sem_out, sem_in):
    cid = lax.axis_index("c")
    sid = lax.axis_index("s")
    grp = sid // NG
    gs = sid % NG
    bt = cid * 2 + grp
    rbase = bt * PBATCH + gs * CH

    def _stage_pts(n):
        pltpu.sync_copy(npts_hbm.at[pl.ds(rbase, n)],
                        fstage.at[pl.ds(0, n)])
        pltpu.sync_copy(fstage.at[pl.ds(0, n)],
                        ftg.at[pl.ds((bt * NPL + C) * BBLK + gs * CH, n)])

    def _chunks(pb, nchk):
        def _chunk(ch, _):
            pltpu.sync_copy(
                feats_hbm.at[pl.ds((rbase + ch * pb) * C, pb * C)],
                fstage.at[pl.ds(0, pb * C)])

            def _tr(v, _):
                vreg = fstage[pl.ds(v * 16, 16)]
                rows = lax.iota(jnp.int32, 16) + (v % 4) * 16
                idx = rows * PB + v // 4
                plsc.store_scatter(trbuf, [idx], vreg)
                return 0

            lax.fori_loop(0, pb * 4, _tr, 0, unroll=4)

            handles = []
            for cch in range(C):
                handles.append(pltpu.async_copy(
                    trbuf.at[pl.ds(cch * PB, PB)],
                    ftg.at[pl.ds((bt * NPL + cch) * BBLK + gs * CH + ch * pb,
                                 PB)],
                    sem_in))
            for h in handles:
                h.wait()
            return 0

        lax.fori_loop(0, nchk, _chunk, 0)

    @pl.when(gs < NG - 1)
    def _():
        _stage_pts(CH)
        _chunks(PB, CH // PB)

    @pl.when(gs == NG - 1)
    def _():
        _stage_pts(LASTN)
        _chunks(PBL, LASTN // PBL)

    iota4 = lax.iota(jnp.int32, 16) * 4
    for j in range(NG):
        n = CH if j < NG - 1 else LASTN
        pltpu.sync_copy(
            coords_hbm.at[pl.ds((bt * PBATCH + j * CH) * 4, n * 4)],
            crow.at[pl.ds(0, n * 4)])

        def _lv(v, _):
            pos = v * 16 + lax.iota(jnp.int32, 16)
            gidx = iota4 + v * 64
            bv = plsc.load_gather(crow, [gidx])
            zv = plsc.load_gather(crow, [gidx + 1])
            yv = plsc.load_gather(crow, [gidx + 2])
            xv = plsc.load_gather(crow, [gidx + 3])
            lin = zv + yv * NX + xv
            ok = (pos < n) & (bv == bt)
            linb[pl.ds(j * CH + v * 16, 16)] = jnp.where(ok, lin, DUMP)
            return 0

        lax.fori_loop(0, (n + 15) // 16, _lv, 0, unroll=4)

        if n < CH:
            def _fill(v, _):
                linb[pl.ds(j * CH + n + v * 16, 16)] = (
                    jnp.zeros((16,), jnp.int32) + DUMP)
                return 0

            lax.fori_loop(0, (CH - n) // 16, _fill, 0)

    plsc.subcore_barrier()

    lo = gs * GS

    def _zero0(v, _):
        str0[pl.ds(v * 16, 16)] = jnp.zeros((16,), jnp.float32)
        str1[pl.ds(v * 16, 16)] = jnp.zeros((16,), jnp.float32)
        return 0

    lax.fori_loop(0, (GS + 8) // 16, _zero0, 0, unroll=8)

    def _plane(k, stripe):
        row_dma = pltpu.make_async_copy(
            ftg.at[pl.ds((bt * NPL + k) * BBLK, BBLK)], rowb, sem_in)
        row_dma.start()

        @pl.when(k >= 2)
        def _():
            pltpu.make_async_copy(
                stripe.at[pl.ds(0, GS)],
                fout.at[pl.ds(lo, GS)],
                sem_out).wait()

        row_dma.wait()

        def _scan(v, _):
            sl = pl.ds(v * 16, 16)
            loc = linb[sl] - lo
            val = rowb[sl]
            ok = (loc >= 0) & (loc < GS)
            plsc.store_scatter(stripe, [jnp.where(ok, loc, GS)], val)
            return 0

        lax.fori_loop(0, BBLK // 16, _scan, 0, unroll=4)

        @pl.when(k < C)
        def _():
            pltpu.async_copy(
                stripe.at[pl.ds(0, GS)],
                fout.at[pl.ds((bt * C + k) * G + lo, GS)],
                sem_out)

        @pl.when(k >= C)
        def _():
            pltpu.async_copy(
                stripe.at[pl.ds(0, GS)],
                pout.at[pl.ds(bt * G + lo, GS)],
                sem_out)

    def _pair(k2, _):
        _plane(k2 * 2, str0)

        @pl.when(k2 * 2 + 1 < NPL)
        def _():
            _plane(k2 * 2 + 1, str1)

        return 0

    lax.fori_loop(0, (NPL + 1) // 2, _pair, 0)

    for stripe in (str0, str1):
        pltpu.make_async_copy(
            stripe.at[pl.ds(0, GS)],
            fout.at[pl.ds(lo, GS)],
            sem_out).wait()


def _make_sc():
    mesh = plsc.VectorSubcoreMesh(core_axis_name="c", subcore_axis_name="s")
    return pl.kernel(
        _sc_body,
        out_type=(
            jax.ShapeDtypeStruct((FEAT_WORDS,), jnp.float32),
            jax.ShapeDtypeStruct((PTS_WORDS,), jnp.float32),
            jax.ShapeDtypeStruct((FTG_WORDS,), jnp.float32),
        ),
        mesh=mesh,
        scratch_types=[
            pltpu.VMEM((4 * CH,), jnp.int32),
            pltpu.VMEM((BBLK,), jnp.int32),
            pltpu.VMEM((BBLK,), jnp.float32),
            pltpu.VMEM((PB * C,), jnp.float32),
            pltpu.VMEM((C * PB,), jnp.float32),
            pltpu.VMEM((GS + 8,), jnp.float32),
            pltpu.VMEM((GS + 8,), jnp.float32),
            pltpu.SemaphoreType.DMA,
            pltpu.SemaphoreType.DMA,
        ],
        compiler_params=pltpu.CompilerParams(needs_layout_passes=False),
    )


def kernel(pillar_features, voxel_coords, voxel_num_points):
    coords = voxel_coords.astype(jnp.int32).reshape(P * 4)
    feats = pillar_features.reshape(P * C)
    fflat, pflat, _ = _make_sc()(coords, feats, voxel_num_points)
    return (fflat.reshape(B, C, NY, NX), pflat.reshape(B, 1, NY, NX))


# speedup vs baseline: 1.0209x; 1.0209x over previous
import jax
import jax.numpy as jnp
from jax import lax
from jax.experimental import pallas as pl
from jax.experimental.pallas import tpu as pltpu
from jax.experimental.pallas import tpu_sc as plsc

NX = 512
NY = 512
G = NX * NY
C = 64
B = 4
P = 40000

NC = 2
NS = 16
NG = 8
CH = 1280
PB_BATCH = P // B
BBLK = NG * CH
PPAD = B * BBLK
GS = G // NG
NPL = C + 1

FEAT_WORDS = B * C * G
PTS_WORDS = B * G

PB_STAGE = (CH // 16) * C
ZB = 2048


def _sc_body(coords_hbm, feats_hbm, npts_hbm, fout, pout,
             crow, pidx, ftT, fstage, zbuf, planeA, planeB, sem_out, sem_in):
    cid = lax.axis_index("c")
    sid = lax.axis_index("s")
    grp = sid // NG
    gs = sid % NG
    bt = cid * 2 + grp
    base = bt * BBLK + gs * CH

    for r in range(4):
        pltpu.sync_copy(coords_hbm.at[r, pl.ds(base, CH)],
                        crow.at[pl.ds(r * CH, CH)])
    pltpu.sync_copy(npts_hbm.at[pl.ds(base, CH)],
                    ftT.at[pl.ds(C * CH, CH)])

    def _idx_body(v, _):
        bv = crow[pl.ds(0 * CH + v * 16, 16)]
        lin = (crow[pl.ds(1 * CH + v * 16, 16)]
               + crow[pl.ds(2 * CH + v * 16, 16)] * NX
               + crow[pl.ds(3 * CH + v * 16, 16)])
        pidx[v // 8, pl.ds((v % 8) * 16, 16)] = jnp.where(bv == bt, lin, G)
        return 0

    lax.fori_loop(0, CH // 16, _idx_body, 0)

    NCHK = 16
    PB = CH // NCHK

    def _chunk(ch, _):
        pltpu.sync_copy(feats_hbm.at[pl.ds((base + ch * PB) * C, PB * C)],
                        fstage)

        def _tr(v, _):
            vreg = fstage[pl.ds(v * 16, 16)]
            p_loc = ch * PB + v // 4
            idx = (lax.iota(jnp.int32, 16) + (v % 4) * 16) * CH + p_loc
            plsc.store_scatter(ftT, [idx], vreg)
            return 0

        lax.fori_loop(0, PB * 4, _tr, 0)
        return 0

    lax.fori_loop(0, NCHK, _chunk, 0)

    def _zb(v, _):
        zbuf[pl.ds(v * 16, 16)] = jnp.zeros((16,), jnp.float32)
        return 0

    lax.fori_loop(0, ZB // 16, _zb, 0)

    stripe_sl = pl.ds(gs * GS, GS)

    def _for_group(fn):
        @pl.when(grp == 0)
        def _():
            fn(planeA)

        @pl.when(grp == 1)
        def _():
            fn(planeB)

    def _plane(k, _):
        def _wait(buf):
            pltpu.make_async_copy(
                buf.at[stripe_sl],
                fout.at[pl.ds(gs * GS, GS)],
                sem_out).wait()

        @pl.when(k >= 1)
        def _():
            _for_group(_wait)

        def _zero(buf):
            for zc in range(GS // ZB):
                pltpu.sync_copy(zbuf, buf.at[pl.ds(gs * GS + zc * ZB, ZB)])

        _for_group(_zero)
        plsc.subcore_barrier()

        def _scatter(buf):
            handles = []
            for row in range(10):
                d = pltpu.make_async_copy(
                    ftT.at[pl.ds(k * CH + row * 128, 128)],
                    buf.at[pidx.at[row]],
                    sem_in)
                d.start(add=True)
                handles.append(d)
            for h in handles:
                h.wait()

        _for_group(_scatter)
        plsc.subcore_barrier()

        def _fire(buf):
            @pl.when(k < C)
            def _():
                pltpu.async_copy(
                    buf.at[stripe_sl],
                    fout.at[pl.ds((bt * C + k) * G + gs * GS, GS)],
                    sem_out)

            @pl.when(k >= C)
            def _():
                pltpu.async_copy(
                    buf.at[stripe_sl],
                    pout.at[pl.ds(bt * G + gs * GS, GS)],
                    sem_out)

        _for_group(_fire)
        return 0

    lax.fori_loop(0, NPL, _plane, 0)

    def _drain(buf):
        pltpu.make_async_copy(
            buf.at[stripe_sl],
            fout.at[pl.ds(gs * GS, GS)],
            sem_out).wait()

    _for_group(_drain)


def _make_sc():
    mesh = plsc.VectorSubcoreMesh(core_axis_name="c", subcore_axis_name="s")
    return pl.kernel(
        _sc_body,
        out_type=(
            jax.ShapeDtypeStruct((FEAT_WORDS,), jnp.float32),
            jax.ShapeDtypeStruct((PTS_WORDS,), jnp.float32),
        ),
        mesh=mesh,
        scratch_types=[
            pltpu.VMEM((4 * CH,), jnp.int32),
            pltpu.VMEM((10, 128), jnp.int32),
            pltpu.VMEM(((C + 1) * CH,), jnp.float32),
            pltpu.VMEM((PB_STAGE,), jnp.float32),
            pltpu.VMEM((ZB,), jnp.float32),
            pltpu.VMEM_SHARED((G + 8,), jnp.float32),
            pltpu.VMEM_SHARED((G + 8,), jnp.float32),
            pltpu.SemaphoreType.DMA,
            pltpu.SemaphoreType.DMA,
        ],
        compiler_params=pltpu.CompilerParams(needs_layout_passes=False),
    )


def kernel(pillar_features, voxel_coords, voxel_num_points):
    coords = voxel_coords.astype(jnp.int32).T

    hpad = BBLK - PB_BATCH
    cpad = jnp.broadcast_to(
        jnp.array([[B], [0], [0], [0]], jnp.int32), (4, hpad))
    cparts = []
    fparts = []
    nparts = []
    fpad = jnp.zeros((hpad, C), jnp.float32)
    npad = jnp.zeros((hpad,), jnp.float32)
    for b in range(B):
        lo, hi = b * PB_BATCH, (b + 1) * PB_BATCH
        cparts += [coords[:, lo:hi], cpad]
        fparts += [pillar_features[lo:hi], fpad]
        nparts += [voxel_num_points[lo:hi], npad]
    coords_p = jnp.concatenate(cparts, axis=-1)
    feats_p = jnp.concatenate(fparts, axis=0).reshape(PPAD * C)
    npts_p = jnp.concatenate(nparts, axis=-1)

    fflat, pflat = _make_sc()(coords_p, feats_p, npts_p)
    return (fflat.reshape(B, C, NY, NX), pflat.reshape(B, 1, NY, NX))
